# initial kernel scaffold (unmeasured)
import jax
import jax.numpy as jnp
from jax import lax
from jax.experimental import pallas as pl
from jax.experimental.pallas import tpu as pltpu


def kernel(
    x,
):
    def body(*refs):
        pass

    out_shape = jax.ShapeDtypeStruct(..., jnp.float32)
    return pl.pallas_call(body, out_shape=out_shape)(...)


# baseline (device time: 54174 ns/iter reference)
import jax
import jax.numpy as jnp
from jax import lax
from jax.experimental import pallas as pl
from jax.experimental.pallas import tpu as pltpu


def kernel(x):
    m, n = x.shape

    def body(x_ref, out_ref, send_sem, recv_sem):
        my_x = lax.axis_index("x")
        my_y = lax.axis_index("y")
        my_z = lax.axis_index("z")
        partner = (my_x, my_y, 1 - my_z)

        barrier_sem = pltpu.get_barrier_semaphore()
        pl.semaphore_signal(
            barrier_sem, inc=1,
            device_id=partner, device_id_type=pl.DeviceIdType.MESH,
        )
        pl.semaphore_wait(barrier_sem, 1)

        rdma = pltpu.make_async_remote_copy(
            src_ref=x_ref,
            dst_ref=out_ref.at[pl.ds(my_z * m, m), :],
            send_sem=send_sem,
            recv_sem=recv_sem,
            device_id=partner,
            device_id_type=pl.DeviceIdType.MESH,
        )
        rdma.start()

        out_ref[pl.ds(my_z * m, m), :] = x_ref[...]

        rdma.wait()

    return pl.pallas_call(
        body,
        out_shape=jax.ShapeDtypeStruct((2 * m, n), x.dtype),
        in_specs=[pl.BlockSpec(memory_space=pltpu.VMEM)],
        out_specs=pl.BlockSpec(memory_space=pltpu.VMEM),
        scratch_shapes=[
            pltpu.SemaphoreType.DMA,
            pltpu.SemaphoreType.DMA,
        ],
        compiler_params=pltpu.CompilerParams(collective_id=0),
    )(x)


# device time: 35209 ns/iter; 1.5386x vs baseline; 1.5386x over previous
import jax
import jax.numpy as jnp
from jax import lax
from jax.experimental import pallas as pl
from jax.experimental.pallas import tpu as pltpu


def kernel(x):
    m, n = x.shape
    q = m // 4

    def body(x_ref, out_ref, send_sems, recv_sems):
        my_x = lax.axis_index("x")
        my_y = lax.axis_index("y")
        my_z = lax.axis_index("z")
        z_nbr = (my_x, my_y, 1 - my_z)
        x_nbr = (1 - my_x, my_y, my_z)
        y_nbr = (my_x, 1 - my_y, my_z)

        q_own = 2 * my_x + my_y
        q_diag = 2 * (1 - my_x) + (1 - my_y)

        mine = my_z * m
        foreign = (1 - my_z) * m

        barrier_sem = pltpu.get_barrier_semaphore()
        for nbr in (z_nbr, x_nbr, y_nbr):
            pl.semaphore_signal(
                barrier_sem, inc=1,
                device_id=nbr, device_id_type=pl.DeviceIdType.MESH,
            )
        pl.semaphore_wait(barrier_sem, 3)

        z1 = pltpu.make_async_remote_copy(
            src_ref=x_ref.at[pl.ds(q_own * q, q), :],
            dst_ref=out_ref.at[pl.ds(mine + q_own * q, q), :],
            send_sem=send_sems.at[0], recv_sem=recv_sems.at[0],
            device_id=z_nbr, device_id_type=pl.DeviceIdType.MESH,
        )
        z2 = pltpu.make_async_remote_copy(
            src_ref=x_ref.at[pl.ds(q_diag * q, q), :],
            dst_ref=out_ref.at[pl.ds(mine + q_diag * q, q), :],
            send_sem=send_sems.at[1], recv_sem=recv_sems.at[1],
            device_id=z_nbr, device_id_type=pl.DeviceIdType.MESH,
        )
        z1.start()
        z2.start()

        out_ref[pl.ds(mine, m), :] = x_ref[...]

        z1.wait_recv()
        xf = pltpu.make_async_remote_copy(
            src_ref=out_ref.at[pl.ds(foreign + q_own * q, q), :],
            dst_ref=out_ref.at[pl.ds(foreign + q_own * q, q), :],
            send_sem=send_sems.at[2], recv_sem=recv_sems.at[2],
            device_id=x_nbr, device_id_type=pl.DeviceIdType.MESH,
        )
        yf = pltpu.make_async_remote_copy(
            src_ref=out_ref.at[pl.ds(foreign + q_own * q, q), :],
            dst_ref=out_ref.at[pl.ds(foreign + q_own * q, q), :],
            send_sem=send_sems.at[3], recv_sem=recv_sems.at[3],
            device_id=y_nbr, device_id_type=pl.DeviceIdType.MESH,
        )
        xf.start()
        yf.start()

        z2.wait_recv()
        xf.wait_recv()
        yf.wait_recv()
        z1.wait_send()
        z2.wait_send()
        xf.wait_send()
        yf.wait_send()

    return pl.pallas_call(
        body,
        out_shape=jax.ShapeDtypeStruct((2 * m, n), x.dtype),
        in_specs=[pl.BlockSpec(memory_space=pltpu.VMEM)],
        out_specs=pl.BlockSpec(memory_space=pltpu.VMEM),
        scratch_shapes=[
            pltpu.SemaphoreType.DMA((4,)),
            pltpu.SemaphoreType.DMA((4,)),
        ],
        compiler_params=pltpu.CompilerParams(collective_id=0),
    )(x)


# device time: 32143 ns/iter; 1.6854x vs baseline; 1.0954x over previous
import jax
import jax.numpy as jnp
from jax import lax
from jax.experimental import pallas as pl
from jax.experimental.pallas import tpu as pltpu

NCH = 4
TA, TB, TC = 168, 168, 176


def kernel(x):
    m, n = x.shape
    q = m // 4
    ch = q // NCH

    def body(x_ref, out_ref, send_sems, recv_sems, copy_sem):
        my_x = lax.axis_index("x")
        my_y = lax.axis_index("y")
        my_z = lax.axis_index("z")
        z_nbr = (my_x, my_y, 1 - my_z)
        x_nbr = (1 - my_x, my_y, my_z)
        y_nbr = (my_x, 1 - my_y, my_z)

        qo = 2 * my_x + my_y
        qd = 2 * (1 - my_x) + (1 - my_y)
        qx = 2 * (1 - my_x) + my_y
        qy = 2 * my_x + (1 - my_y)

        mine = my_z * m
        forn = (1 - my_z) * m

        def rc(rows, nrows, sem_i, dev):
            return pltpu.make_async_remote_copy(
                src_ref=out_ref.at[pl.ds(rows, nrows), :],
                dst_ref=out_ref.at[pl.ds(rows, nrows), :],
                send_sem=send_sems.at[sem_i], recv_sem=recv_sems.at[sem_i],
                device_id=dev, device_id_type=pl.DeviceIdType.MESH,
            )

        barrier_sem = pltpu.get_barrier_semaphore()
        for nbr in (z_nbr, x_nbr, y_nbr):
            pl.semaphore_signal(
                barrier_sem, inc=1,
                device_id=nbr, device_id_type=pl.DeviceIdType.MESH,
            )
        pl.semaphore_wait(barrier_sem, 3)

        z_own = []
        for i in range(NCH):
            off = qo * q + i * ch
            z_own.append(pltpu.make_async_remote_copy(
                src_ref=x_ref.at[pl.ds(off, ch), :],
                dst_ref=out_ref.at[pl.ds(mine + off, ch), :],
                send_sem=send_sems.at[i], recv_sem=recv_sems.at[i],
                device_id=z_nbr, device_id_type=pl.DeviceIdType.MESH,
            ))
        z_diag = pltpu.make_async_remote_copy(
            src_ref=x_ref.at[pl.ds(qd * q, TA), :],
            dst_ref=out_ref.at[pl.ds(mine + qd * q, TA), :],
            send_sem=send_sems.at[NCH], recv_sem=recv_sems.at[NCH],
            device_id=z_nbr, device_id_type=pl.DeviceIdType.MESH,
        )
        for r in z_own:
            r.start()
        z_diag.start()

        cp = pltpu.make_async_copy(
            x_ref, out_ref.at[pl.ds(mine, m), :], copy_sem)
        cp.start()

        xf_own, yf_own = [], []
        for i in range(NCH):
            rows = forn + qo * q + i * ch
            z_own[i].wait_recv()
            xf = rc(rows, ch, NCH + 1 + i, x_nbr)
            yf = rc(rows, ch, 2 * NCH + 1 + i, y_nbr)
            xf.start()
            yf.start()
            xf_own.append(xf)
            yf_own.append(yf)

        xf_own[1].wait_recv()
        xf_own[2].wait_recv()
        yf_third = rc(forn + qx * q + TA, TB, 3 * NCH + 1, y_nbr)
        yf_third.start()

        yf_own[2].wait_recv()
        yf_own[3].wait_recv()
        xf_third = rc(forn + qy * q + TA + TB, TC, 3 * NCH + 2, x_nbr)
        xf_third.start()

        z_diag.wait_recv()
        xf_own[0].wait_recv()
        xf_own[3].wait_recv()
        yf_own[0].wait_recv()
        yf_own[1].wait_recv()
        xf_third.wait_recv()
        yf_third.wait_recv()

        for r in z_own:
            r.wait_send()
        z_diag.wait_send()
        for r in xf_own:
            r.wait_send()
        for r in yf_own:
            r.wait_send()
        xf_third.wait_send()
        yf_third.wait_send()
        cp.wait()

    n_sems = 3 * NCH + 3
    return pl.pallas_call(
        body,
        out_shape=jax.ShapeDtypeStruct((2 * m, n), x.dtype),
        in_specs=[pl.BlockSpec(memory_space=pltpu.VMEM)],
        out_specs=pl.BlockSpec(memory_space=pltpu.VMEM),
        scratch_shapes=[
            pltpu.SemaphoreType.DMA((n_sems,)),
            pltpu.SemaphoreType.DMA((n_sems,)),
            pltpu.SemaphoreType.DMA,
        ],
        compiler_params=pltpu.CompilerParams(collective_id=0),
    )(x)


# device time: 31016 ns/iter; 1.7466x vs baseline; 1.0363x over previous
import jax
import jax.numpy as jnp
from jax import lax
from jax.experimental import pallas as pl
from jax.experimental.pallas import tpu as pltpu

CHUNKS = (32, 64, 128, 160, 128)
OFFS = (0, 32, 96, 224, 384)
NCH = len(CHUNKS)
TB, TC, TA = 120, 120, 272


def kernel(x):
    m, n = x.shape
    q = m // 4

    def body(x_ref, out_ref, comm_ref, send_sems, recv_sems, copy_sems):
        my_x = lax.axis_index("x")
        my_y = lax.axis_index("y")
        my_z = lax.axis_index("z")
        z_nbr = (my_x, my_y, 1 - my_z)
        x_nbr = (1 - my_x, my_y, my_z)
        y_nbr = (my_x, 1 - my_y, my_z)

        qo = 2 * my_x + my_y
        qd = 2 * (1 - my_x) + (1 - my_y)
        qx = 2 * (1 - my_x) + my_y
        qy = 2 * my_x + (1 - my_y)

        mine = my_z * m
        forn = (1 - my_z) * m

        def rc(rows, nrows, sem_i, dev):
            return pltpu.make_async_remote_copy(
                src_ref=comm_ref.at[pl.ds(rows, nrows), :],
                dst_ref=comm_ref.at[pl.ds(rows, nrows), :],
                send_sem=send_sems.at[sem_i], recv_sem=recv_sems.at[sem_i],
                device_id=dev, device_id_type=pl.DeviceIdType.MESH,
            )

        def copy_out(rows, nrows, sem_i):
            cp = pltpu.make_async_copy(
                comm_ref.at[pl.ds(rows, nrows), :],
                out_ref.at[pl.ds(forn + rows, nrows), :],
                copy_sems.at[sem_i],
            )
            cp.start()
            return cp

        barrier_sem = pltpu.get_barrier_semaphore()
        for nbr in (z_nbr, x_nbr, y_nbr):
            pl.semaphore_signal(
                barrier_sem, inc=1,
                device_id=nbr, device_id_type=pl.DeviceIdType.MESH,
            )
        pl.semaphore_wait(barrier_sem, 3)

        z_own = []
        for i in range(NCH):
            off = qo * q + OFFS[i]
            z_own.append(pltpu.make_async_remote_copy(
                src_ref=x_ref.at[pl.ds(off, CHUNKS[i]), :],
                dst_ref=comm_ref.at[pl.ds(off, CHUNKS[i]), :],
                send_sem=send_sems.at[i], recv_sem=recv_sems.at[i],
                device_id=z_nbr, device_id_type=pl.DeviceIdType.MESH,
            ))
        z_diag = pltpu.make_async_remote_copy(
            src_ref=x_ref.at[pl.ds(qd * q + TB + TC, TA), :],
            dst_ref=comm_ref.at[pl.ds(qd * q + TB + TC, TA), :],
            send_sem=send_sems.at[NCH], recv_sem=recv_sems.at[NCH],
            device_id=z_nbr, device_id_type=pl.DeviceIdType.MESH,
        )
        for r in z_own:
            r.start()
        z_diag.start()

        cp_mine = pltpu.make_async_copy(
            x_ref, out_ref.at[pl.ds(mine, m), :], copy_sems.at[0])
        cp_mine.start()

        xf_own, yf_own = [], []
        for i in range(NCH):
            rows = qo * q + OFFS[i]
            z_own[i].wait_recv()
            xf = rc(rows, CHUNKS[i], NCH + 1 + i, x_nbr)
            yf = rc(rows, CHUNKS[i], 2 * NCH + 1 + i, y_nbr)
            xf.start()
            yf.start()
            xf_own.append(xf)
            yf_own.append(yf)
        cp_qo = copy_out(qo * q, q, 1)

        xf_own[0].wait_recv()
        xf_own[1].wait_recv()
        xf_own[2].wait_recv()
        yf_third = rc(qx * q, TB, 3 * NCH + 1, y_nbr)
        yf_third.start()

        yf_own[2].wait_recv()
        yf_own[3].wait_recv()
        xf_third = rc(qy * q + TB, TC, 3 * NCH + 2, x_nbr)
        xf_third.start()

        xf_own[3].wait_recv()
        xf_own[4].wait_recv()
        cp_qx = copy_out(qx * q, q, 2)
        yf_own[0].wait_recv()
        yf_own[1].wait_recv()
        yf_own[4].wait_recv()
        cp_qy = copy_out(qy * q, q, 3)
        z_diag.wait_recv()
        cp_qdA = copy_out(qd * q + TB + TC, TA, 4)
        yf_third.wait_recv()
        cp_qdB = copy_out(qd * q, TB, 5)
        xf_third.wait_recv()
        cp_qdC = copy_out(qd * q + TB, TC, 6)

        for r in z_own:
            r.wait_send()
        z_diag.wait_send()
        for r in xf_own:
            r.wait_send()
        for r in yf_own:
            r.wait_send()
        xf_third.wait_send()
        yf_third.wait_send()
        cp_mine.wait()
        cp_qo.wait()
        cp_qx.wait()
        cp_qy.wait()
        cp_qdA.wait()
        cp_qdB.wait()
        cp_qdC.wait()

    n_sems = 3 * NCH + 3
    return pl.pallas_call(
        body,
        out_shape=jax.ShapeDtypeStruct((2 * m, n), x.dtype),
        in_specs=[pl.BlockSpec(memory_space=pltpu.MemorySpace.HBM)],
        out_specs=pl.BlockSpec(memory_space=pltpu.MemorySpace.HBM),
        scratch_shapes=[
            pltpu.VMEM((m, n), x.dtype),
            pltpu.SemaphoreType.DMA((n_sems,)),
            pltpu.SemaphoreType.DMA((n_sems,)),
            pltpu.SemaphoreType.DMA((7,)),
        ],
        compiler_params=pltpu.CompilerParams(collective_id=0),
    )(x)
